# baseline (speedup 1.0000x reference)
import jax
import jax.numpy as jnp
from jax.experimental import pallas as pl
from jax.experimental.pallas import tpu as pltpu

_N = 2048
_D = 64
_PB = 2


def _gcn_body(x_ref, adj_ref, w1_ref, b1_ref, w2_ref, b2_ref,
              v1_ref, c1_ref, v2_ref, c2_ref, fcw_ref, fcb_ref, out_ref):
    def gconv(b, h, w1, b1, w2, b2):
        agg = jnp.dot(adj_ref[b], h.astype(jnp.bfloat16),
                      preferred_element_type=jnp.float32)
        return (jnp.dot(h, w1, preferred_element_type=jnp.float32) + b1
                + jnp.dot(agg, w2, preferred_element_type=jnp.float32) + b2)

    hs = [x_ref[b] for b in range(_PB)]
    hs = [gconv(b, hs[b], w1_ref[0], b1_ref[0], w2_ref[0], b2_ref[0])
          for b in range(_PB)]
    for r in range(6):
        i, j = 1 + 2 * r, 2 + 2 * r
        o1 = [jnp.maximum(
            gconv(b, hs[b], w1_ref[i], b1_ref[i], w2_ref[i], b2_ref[i]), 0.0)
            for b in range(_PB)]
        hs = [jnp.maximum(
            gconv(b, o1[b], w1_ref[j], b1_ref[j], w2_ref[j], b2_ref[j])
            + hs[b], 0.0)
            for b in range(_PB)]
    gs = [gconv(b, hs[b], v1_ref[0], c1_ref[0], v2_ref[0], c2_ref[0])
          for b in range(_PB)]
    for b in range(_PB):
        out_ref[b] = (jnp.dot(gs[b], fcw_ref[...],
                              preferred_element_type=jnp.float32)
                      + fcb_ref[...])


def kernel(x, adj, params):
    B = x.shape[0]
    adj = adj.astype(jnp.bfloat16)
    L = params["layers"]
    w1 = jnp.stack([l["W1"] for l in L[:13]])
    b1 = jnp.stack([l["b1"] for l in L[:13]])[:, None, :]
    w2 = jnp.stack([l["W2"] for l in L[:13]])
    b2 = jnp.stack([l["b2"] for l in L[:13]])[:, None, :]
    v1 = L[13]["W1"][None]
    c1 = L[13]["b1"][None, None, :]
    v2 = L[13]["W2"][None]
    c2 = L[13]["b2"][None, None, :]
    fcw = params["fcW"]
    fcb = params["fcb"][None, :]

    full = lambda s: pl.BlockSpec(s, lambda g: (0,) * len(s))
    return pl.pallas_call(
        _gcn_body,
        grid=(B // _PB,),
        in_specs=[
            pl.BlockSpec((_PB, _N, _D), lambda g: (g, 0, 0)),
            pl.BlockSpec((_PB, _N, _N), lambda g: (g, 0, 0)),
            full((13, _D, _D)), full((13, 1, _D)),
            full((13, _D, _D)), full((13, 1, _D)),
            full((1, _D, 32)), full((1, 1, 32)),
            full((1, _D, 32)), full((1, 1, 32)),
            full((32, 2)), full((1, 2)),
        ],
        out_specs=pl.BlockSpec((_PB, _N, 2), lambda g: (g, 0, 0)),
        out_shape=jax.ShapeDtypeStruct((B, _N, 2), jnp.float32),
        compiler_params=pltpu.CompilerParams(
            dimension_semantics=("arbitrary",),
            vmem_limit_bytes=110 * 1024 * 1024,
        ),
    )(x, adj, w1, b1, w2, b2, v1, c1, v2, c2, fcw, fcb)


# speedup vs baseline: 1.0337x; 1.0337x over previous
import jax
import jax.numpy as jnp
from jax.experimental import pallas as pl
from jax.experimental.pallas import tpu as pltpu

_N = 2048
_D = 64


def _gcn_body(x_ref, adj_ref, w1_ref, b1_ref, w2_ref, b2_ref,
              v1_ref, c1_ref, v2_ref, c2_ref, fcw_ref, fcb_ref, out_ref):
    adj = adj_ref[0].astype(jnp.bfloat16)
    h = x_ref[0]

    def gconv(h, w1, b1, w2, b2):
        agg = jnp.dot(adj, h.astype(jnp.bfloat16),
                      preferred_element_type=jnp.float32)
        return (jnp.dot(h, w1, preferred_element_type=jnp.float32) + b1
                + jnp.dot(agg, w2, preferred_element_type=jnp.float32) + b2)

    h = gconv(h, w1_ref[0], b1_ref[0], w2_ref[0], b2_ref[0])
    for r in range(6):
        i, j = 1 + 2 * r, 2 + 2 * r
        o1 = jnp.maximum(
            gconv(h, w1_ref[i], b1_ref[i], w2_ref[i], b2_ref[i]), 0.0)
        h = jnp.maximum(
            gconv(o1, w1_ref[j], b1_ref[j], w2_ref[j], b2_ref[j]) + h, 0.0)
    g = gconv(h, v1_ref[0], c1_ref[0], v2_ref[0], c2_ref[0])
    out_ref[0] = (jnp.dot(g, fcw_ref[...], preferred_element_type=jnp.float32)
                  + fcb_ref[...])


def kernel(x, adj, params):
    B = x.shape[0]
    L = params["layers"]
    w1 = jnp.stack([l["W1"] for l in L[:13]])
    b1 = jnp.stack([l["b1"] for l in L[:13]])[:, None, :]
    w2 = jnp.stack([l["W2"] for l in L[:13]])
    b2 = jnp.stack([l["b2"] for l in L[:13]])[:, None, :]
    v1 = L[13]["W1"][None]
    c1 = L[13]["b1"][None, None, :]
    v2 = L[13]["W2"][None]
    c2 = L[13]["b2"][None, None, :]
    fcw = params["fcW"]
    fcb = params["fcb"][None, :]

    full = lambda s: pl.BlockSpec(s, lambda b: (0,) * len(s))
    grid_spec = pl.GridSpec(
        grid=(B,),
        in_specs=[
            pl.BlockSpec((1, _N, _D), lambda b: (b, 0, 0)),
            pl.BlockSpec((1, _N, _N), lambda b: (b, 0, 0)),
            full((13, _D, _D)), full((13, 1, _D)),
            full((13, _D, _D)), full((13, 1, _D)),
            full((1, _D, 32)), full((1, 1, 32)),
            full((1, _D, 32)), full((1, 1, 32)),
            full((32, 2)), full((1, 2)),
        ],
        out_specs=pl.BlockSpec((1, _N, 2), lambda b: (b, 0, 0)),
    )
    return pl.pallas_call(
        _gcn_body,
        grid_spec=grid_spec,
        out_shape=jax.ShapeDtypeStruct((B, _N, 2), jnp.float32),
        compiler_params=pltpu.CompilerParams(
            dimension_semantics=("parallel",),
            vmem_limit_bytes=100 * 1024 * 1024,
        ),
    )(x, adj, w1, b1, w2, b2, v1, c1, v2, c2, fcw, fcb)


# speedup vs baseline: 1.0680x; 1.0332x over previous
import jax
import jax.numpy as jnp
from jax.experimental import pallas as pl
from jax.experimental.pallas import tpu as pltpu

_N = 2048
_D = 64


def _gcn_body(x_ref, adj_ref, w_ref, b_ref, v_ref, c_ref, fcw_ref, fcb_ref,
              out_ref):
    adj = adj_ref[0]

    def gconv(h, w, b):
        agg = jnp.dot(adj, h, preferred_element_type=jnp.float32)
        u = jnp.concatenate([h, agg], axis=1)
        return jnp.dot(u, w, preferred_element_type=jnp.float32) + b

    h = gconv(x_ref[0], w_ref[0], b_ref[0])
    for r in range(6):
        i, j = 1 + 2 * r, 2 + 2 * r
        o1 = jnp.maximum(gconv(h, w_ref[i], b_ref[i]), 0.0)
        h = jnp.maximum(gconv(o1, w_ref[j], b_ref[j]) + h, 0.0)
    g = gconv(h, v_ref[0], c_ref[0])
    out_ref[0] = (jnp.dot(g, fcw_ref[...], preferred_element_type=jnp.float32)
                  + fcb_ref[...])


def kernel(x, adj, params):
    B = x.shape[0]
    L = params["layers"]
    w = jnp.stack([jnp.concatenate([l["W1"], l["W2"]], axis=0)
                   for l in L[:13]])
    b = jnp.stack([l["b1"] + l["b2"] for l in L[:13]])[:, None, :]
    v = jnp.concatenate([L[13]["W1"], L[13]["W2"]], axis=0)[None]
    c = (L[13]["b1"] + L[13]["b2"])[None, None, :]
    fcw = params["fcW"]
    fcb = params["fcb"][None, :]

    full = lambda s: pl.BlockSpec(s, lambda g: (0,) * len(s))
    grid_spec = pl.GridSpec(
        grid=(B,),
        in_specs=[
            pl.BlockSpec((1, _N, _D), lambda g: (g, 0, 0)),
            pl.BlockSpec((1, _N, _N), lambda g: (g, 0, 0)),
            full((13, 2 * _D, _D)), full((13, 1, _D)),
            full((1, 2 * _D, 32)), full((1, 1, 32)),
            full((32, 2)), full((1, 2)),
        ],
        out_specs=pl.BlockSpec((1, _N, 2), lambda g: (g, 0, 0)),
    )
    return pl.pallas_call(
        _gcn_body,
        grid_spec=grid_spec,
        out_shape=jax.ShapeDtypeStruct((B, _N, 2), jnp.float32),
        compiler_params=pltpu.CompilerParams(
            dimension_semantics=("arbitrary",),
            vmem_limit_bytes=100 * 1024 * 1024,
        ),
    )(x, adj, w, b, v, c, fcw, fcb)


# speedup vs baseline: 1.0754x; 1.0070x over previous
import jax
import jax.numpy as jnp
from jax.experimental import pallas as pl
from jax.experimental.pallas import tpu as pltpu

_N = 2048
_D = 64


def _gcn_body(x_ref, adj_ref, w_ref, b_ref, v_ref, c_ref, out_ref):
    adj = adj_ref[0]

    def gconv(h, w, b):
        agg = jnp.dot(adj, h, preferred_element_type=jnp.float32)
        u = jnp.concatenate([h, agg], axis=1)
        return jnp.dot(u, w, preferred_element_type=jnp.float32) + b

    h = gconv(x_ref[0], w_ref[0], b_ref[0])
    for r in range(6):
        i, j = 1 + 2 * r, 2 + 2 * r
        o1 = jnp.maximum(gconv(h, w_ref[i], b_ref[i]), 0.0)
        h = jnp.maximum(gconv(o1, w_ref[j], b_ref[j]) + h, 0.0)
    out_ref[0] = gconv(h, v_ref[0], c_ref[0])


def kernel(x, adj, params):
    B = x.shape[0]
    L = params["layers"]
    w = jnp.stack([jnp.concatenate([l["W1"], l["W2"]], axis=0)
                   for l in L[:13]])
    b = jnp.stack([l["b1"] + l["b2"] for l in L[:13]])[:, None, :]
    v13 = jnp.concatenate([L[13]["W1"], L[13]["W2"]], axis=0)
    b13 = L[13]["b1"] + L[13]["b2"]
    v = (v13 @ params["fcW"])[None]
    c = (b13 @ params["fcW"] + params["fcb"])[None, None, :]

    full = lambda s: pl.BlockSpec(s, lambda g: (0,) * len(s))
    grid_spec = pl.GridSpec(
        grid=(B,),
        in_specs=[
            pl.BlockSpec((1, _N, _D), lambda g: (g, 0, 0)),
            pl.BlockSpec((1, _N, _N), lambda g: (g, 0, 0)),
            full((13, 2 * _D, _D)), full((13, 1, _D)),
            full((1, 2 * _D, 2)), full((1, 1, 2)),
        ],
        out_specs=pl.BlockSpec((1, _N, 2), lambda g: (g, 0, 0)),
    )
    return pl.pallas_call(
        _gcn_body,
        grid_spec=grid_spec,
        out_shape=jax.ShapeDtypeStruct((B, _N, 2), jnp.float32),
        compiler_params=pltpu.CompilerParams(
            dimension_semantics=("arbitrary",),
            vmem_limit_bytes=100 * 1024 * 1024,
        ),
    )(x, adj, w, b, v, c)
